# Initial kernel scaffold; baseline (speedup 1.0000x reference)
import functools

import jax
import jax.numpy as jnp
from jax import lax
from jax.experimental import pallas as pl
from jax.experimental.pallas import tpu as pltpu
from jax.experimental.pallas import tpu_sc as plsc

N = 10000
M = 320000
F = 128
U = 128

NC, NS, L = 2, 16, 16
NW = NC * NS
EPW = M // NW
CH = 80
NCH = EPW // CH
NPAD = 10240
ZCH = NPAD // NS
RPT = N // NS

_mesh = plsc.VectorSubcoreMesh(
    core_axis_name="c", subcore_axis_name="s", num_cores=NC, num_subcores=NS
)


def _tables_body(x_ref, wcat_ref, bcat_ref, wn_ref, sin_ref, sout_ref):
    x = x_ref[...]
    wn_ref[...] = (
        jnp.dot(x, wcat_ref[:, 0:U], preferred_element_type=jnp.float32)
        + bcat_ref[0, :]
    )
    sin_ref[...] = jnp.dot(x, wcat_ref[:, U : 2 * U], preferred_element_type=jnp.float32)
    sout_ref[...] = (
        jnp.dot(x, wcat_ref[:, 2 * U : 3 * U], preferred_element_type=jnp.float32)
        + bcat_ref[1, :]
    )


_TBLK = 2000
_tables = pl.pallas_call(
    _tables_body,
    grid=(N // _TBLK,),
    in_specs=[
        pl.BlockSpec((_TBLK, F), lambda i: (i, 0)),
        pl.BlockSpec((F, 3 * U), lambda i: (0, 0)),
        pl.BlockSpec((2, U), lambda i: (0, 0)),
    ],
    out_specs=[pl.BlockSpec((_TBLK, U), lambda i: (i, 0))] * 3,
    out_shape=[jax.ShapeDtypeStruct((N, U), jnp.float32)] * 3,
)


def _passA_body(
    recv_h, send_h, sin_h, sout_h, wal_h,
    ea_h, psum_h,
    ridx, sidx, rin, rout, eav, wv, zv, asum_sh, sem1, sem2,
):
    cid = lax.axis_index("c")
    sid = lax.axis_index("s")
    wid = sid * NC + cid
    pltpu.sync_copy(wal_h, wv)

    def zbody(i, c):
        zv[pl.ds(i * L, L)] = jnp.zeros((L,), jnp.float32)
        return c

    lax.fori_loop(0, ZCH // L, zbody, 0)
    pltpu.sync_copy(zv, asum_sh.at[pl.ds(sid * ZCH, ZCH)])
    plsc.subcore_barrier()

    base0 = wid * EPW

    def chunk(t, c):
        base = base0 + t * CH
        pltpu.sync_copy(recv_h.at[pl.ds(base, CH)], ridx)
        pltpu.sync_copy(send_h.at[pl.ds(base, CH)], sidx)
        cp1 = pltpu.async_copy(sin_h.at[ridx], rin, sem1)
        cp2 = pltpu.async_copy(sout_h.at[sidx], rout, sem2)
        cp1.wait()
        cp2.wait()

        def edge(e, cc):
            acc = jnp.zeros((L,), jnp.float32)
            for k in range(U // L):
                s = pl.ds(k * L, L)
                z = rin[e, s] + rout[e, s]
                z = jnp.maximum(z, 0.2 * z)
                acc = acc + z * wv[s]
            eav[e] = jnp.sum(acc)
            return cc

        lax.fori_loop(0, CH, edge, 0)

        def expb(k, cc):
            s = pl.ds(k * L, L)
            eav[s] = jnp.exp(eav[s])
            return cc

        lax.fori_loop(0, CH // L, expb, 0)
        pltpu.sync_copy(eav, ea_h.at[pl.ds(base, CH)])
        pltpu.sync_copy(eav, asum_sh.at[ridx], add=True)
        return c

    lax.fori_loop(0, NCH, chunk, 0)
    plsc.subcore_barrier()

    @pl.when(sid == 0)
    def _():
        pltpu.sync_copy(asum_sh.at[pl.ds(0, N)], psum_h.at[cid])


_passA = pl.kernel(
    _passA_body,
    out_type=[
        jax.ShapeDtypeStruct((M,), jnp.float32),
        jax.ShapeDtypeStruct((NC, N), jnp.float32),
    ],
    mesh=_mesh,
    scratch_types=[
        pltpu.VMEM((CH,), jnp.int32),
        pltpu.VMEM((CH,), jnp.int32),
        pltpu.VMEM((CH, U), jnp.float32),
        pltpu.VMEM((CH, U), jnp.float32),
        pltpu.VMEM((CH,), jnp.float32),
        pltpu.VMEM((U,), jnp.float32),
        pltpu.VMEM((ZCH,), jnp.float32),
        pltpu.VMEM_SHARED((NPAD,), jnp.float32),
        pltpu.SemaphoreType.DMA,
        pltpu.SemaphoreType.DMA,
    ],
)


_ZR = 125


def _passB_body(
    recv_h, send_h, ea_h, p0_h, p1_h, wn_h,
    pout_h,
    ridx, sidx, eav, s0v, s1v, rows, zrows, out_sh, sem1, sem2, sem3,
):
    cid = lax.axis_index("c")
    sid = lax.axis_index("s")
    wid = sid * NC + cid

    def zbody(i, c):
        zrows[i // (U // L), pl.ds((i % (U // L)) * L, L)] = jnp.zeros(
            (L,), jnp.float32
        )
        return c

    lax.fori_loop(0, _ZR * (U // L), zbody, 0)
    for i in range(RPT // _ZR):
        pltpu.sync_copy(zrows, out_sh.at[pl.ds(sid * RPT + i * _ZR, _ZR)])
    plsc.subcore_barrier()

    base0 = wid * EPW

    def chunk(t, c):
        base = base0 + t * CH
        pltpu.sync_copy(recv_h.at[pl.ds(base, CH)], ridx)
        pltpu.sync_copy(send_h.at[pl.ds(base, CH)], sidx)
        cp1 = pltpu.async_copy(wn_h.at[sidx], rows, sem1)
        cp2 = pltpu.async_copy(p0_h.at[ridx], s0v, sem2)
        cp3 = pltpu.async_copy(p1_h.at[ridx], s1v, sem3)
        pltpu.sync_copy(ea_h.at[pl.ds(base, CH)], eav)
        cp2.wait()
        cp3.wait()

        def ab(k, cc):
            s = pl.ds(k * L, L)
            eav[s] = eav[s] / jnp.maximum(s0v[s] + s1v[s], 1e-16)
            return cc

        lax.fori_loop(0, CH // L, ab, 0)
        cp1.wait()

        def edge(e, cc):
            a = eav[e]
            for k in range(U // L):
                s = pl.ds(k * L, L)
                rows[e, s] = rows[e, s] * a
            return cc

        lax.fori_loop(0, CH, edge, 0)
        pltpu.sync_copy(rows, out_sh.at[ridx], add=True)
        return c

    lax.fori_loop(0, NCH, chunk, 0)
    plsc.subcore_barrier()
    for i in range(RPT // _ZR):
        r = sid * RPT + i * _ZR
        pltpu.sync_copy(out_sh.at[pl.ds(r, _ZR)], pout_h.at[cid, pl.ds(r, _ZR)])


_passB = pl.kernel(
    _passB_body,
    out_type=jax.ShapeDtypeStruct((NC, N, U), jnp.float32),
    mesh=_mesh,
    scratch_types=[
        pltpu.VMEM((CH,), jnp.int32),
        pltpu.VMEM((CH,), jnp.int32),
        pltpu.VMEM((CH,), jnp.float32),
        pltpu.VMEM((CH,), jnp.float32),
        pltpu.VMEM((CH,), jnp.float32),
        pltpu.VMEM((CH, U), jnp.float32),
        pltpu.VMEM((_ZR, U), jnp.float32),
        pltpu.VMEM_SHARED((N, U), jnp.float32),
        pltpu.SemaphoreType.DMA,
        pltpu.SemaphoreType.DMA,
        pltpu.SemaphoreType.DMA,
    ],
)


def _combine_body(p_ref, out_ref):
    x = p_ref[0] + p_ref[1]
    out_ref[...] = jnp.where(x > 0, x, jnp.expm1(x))


_combine = pl.pallas_call(
    _combine_body,
    grid=(N // _TBLK,),
    in_specs=[pl.BlockSpec((NC, _TBLK, U), lambda i: (0, i, 0))],
    out_specs=pl.BlockSpec((_TBLK, U), lambda i: (i, 0)),
    out_shape=jax.ShapeDtypeStruct((N, U), jnp.float32),
)


def kernel(node, edge, edge_index, W_lin, b_lin, W_att, b_att, w_alpha):
    recv = edge_index[:, 0]
    send = edge_index[:, 1]
    wcat = jnp.concatenate([W_lin, W_att[:F], W_att[F:]], axis=1)
    bcat = jnp.stack([b_lin, b_att])
    wn, sin, sout = _tables(node, wcat, bcat)
    ea, psum = _passA(recv, send, sin, sout, w_alpha[:, 0])
    pout = _passB(recv, send, ea, psum[0], psum[1], wn)
    return _combine(pout)


# speedup vs baseline: 8.3976x; 8.3976x over previous
import jax
import jax.numpy as jnp
from jax import lax
from jax.experimental import pallas as pl
from jax.experimental.pallas import tpu as pltpu
from jax.experimental.pallas import tpu_sc as plsc

N = 10000
M = 320000
F = 128
U = 128

NC, NS, L = 2, 16, 16
NW = NC * NS
EPW = M // NW
CH = 80
NCH = EPW // CH
NPAD = 10240
ZCH = NPAD // NS
RPT = NPAD // NS
ZR = 128

_mesh = plsc.VectorSubcoreMesh(
    core_axis_name="c", subcore_axis_name="s", num_cores=NC, num_subcores=NS
)


def _tables_body(x_ref, wcat_ref, bcat_ref, wn_ref, sin_ref, sout_ref):
    x = x_ref[...]
    wn_ref[...] = (
        jnp.dot(x, wcat_ref[:, 0:U], preferred_element_type=jnp.float32)
        + bcat_ref[0, :]
    )
    sin_ref[...] = jnp.dot(x, wcat_ref[:, U : 2 * U], preferred_element_type=jnp.float32)
    sout_ref[...] = (
        jnp.dot(x, wcat_ref[:, 2 * U : 3 * U], preferred_element_type=jnp.float32)
        + bcat_ref[1, :]
    )


_TBLK = 2000
_tables = pl.pallas_call(
    _tables_body,
    grid=(N // _TBLK,),
    in_specs=[
        pl.BlockSpec((_TBLK, F), lambda i: (i, 0)),
        pl.BlockSpec((F, 3 * U), lambda i: (0, 0)),
        pl.BlockSpec((2, U), lambda i: (0, 0)),
    ],
    out_specs=[pl.BlockSpec((_TBLK, U), lambda i: (i, 0))] * 3,
    out_shape=[jax.ShapeDtypeStruct((N, U), jnp.float32)] * 3,
)


def _passA_body(
    recv_h, send_h, sin_h, sout_h, wal_h,
    ea_h, psum_h,
    ridx, sidx, rin, rout, eav, wv, zv, asum_sh, sem1, sem2,
):
    cid = lax.axis_index("c")
    sid = lax.axis_index("s")
    wid = sid * NC + cid
    pltpu.sync_copy(wal_h, wv)

    def zbody(i, c):
        zv[pl.ds(i * L, L)] = jnp.zeros((L,), jnp.float32)
        return c

    lax.fori_loop(0, ZCH // L, zbody, 0)
    pltpu.sync_copy(zv, asum_sh.at[pl.ds(sid * ZCH, ZCH)])
    plsc.subcore_barrier()

    base0 = wid * EPW
    lanes = lax.iota(jnp.int32, L)

    def chunk(t, c):
        base = base0 + t * CH
        pltpu.sync_copy(recv_h.at[pl.ds(base, CH)], ridx)
        pltpu.sync_copy(send_h.at[pl.ds(base, CH)], sidx)
        cp1 = pltpu.async_copy(sin_h.at[ridx], rin, sem1)
        cp2 = pltpu.async_copy(sout_h.at[sidx], rout, sem2)
        cp1.wait()
        cp2.wait()

        def edge_group(g, cc):
            vals = jnp.zeros((L,), jnp.float32)
            for j in range(L):
                e = g * L + j
                acc = jnp.zeros((L,), jnp.float32)
                for k in range(U // L):
                    s = pl.ds(k * L, L)
                    z = rin[e, s] + rout[e, s]
                    z = jnp.maximum(z, 0.2 * z)
                    acc = acc + z * wv[s]
                tot = acc[0]
                for q in range(1, L):
                    tot = tot + acc[q]
                vals = jnp.where(lanes == j, tot, vals)
            eav[pl.ds(g * L, L)] = jnp.exp(vals)
            return cc

        lax.fori_loop(0, CH // L, edge_group, 0)
        pltpu.sync_copy(eav, ea_h.at[pl.ds(base, CH)])
        pltpu.sync_copy(eav, asum_sh.at[ridx], add=True)
        return c

    lax.fori_loop(0, NCH, chunk, 0)
    plsc.subcore_barrier()

    @pl.when(sid == 0)
    def _():
        pltpu.sync_copy(asum_sh, psum_h.at[pl.ds(cid * NPAD, NPAD)])


_passA = pl.kernel(
    _passA_body,
    out_type=[
        jax.ShapeDtypeStruct((M,), jnp.float32),
        jax.ShapeDtypeStruct((NC * NPAD,), jnp.float32),
    ],
    mesh=_mesh,
    scratch_types=[
        pltpu.VMEM((CH,), jnp.int32),
        pltpu.VMEM((CH,), jnp.int32),
        pltpu.VMEM((CH, U), jnp.float32),
        pltpu.VMEM((CH, U), jnp.float32),
        pltpu.VMEM((CH,), jnp.float32),
        pltpu.VMEM((U,), jnp.float32),
        pltpu.VMEM((ZCH,), jnp.float32),
        pltpu.VMEM_SHARED((NPAD,), jnp.float32),
        pltpu.SemaphoreType.DMA,
        pltpu.SemaphoreType.DMA,
    ],
)


def _passB_body(
    recv_h, send_h, ea_h, p0_h, p1_h, wn_h,
    pout_h,
    ridx, sidx, eav, s0v, s1v, rows, zrows, out_sh, sem1, sem2, sem3,
):
    cid = lax.axis_index("c")
    sid = lax.axis_index("s")
    wid = sid * NC + cid

    def zbody(i, c):
        zrows[i // (U // L), pl.ds((i % (U // L)) * L, L)] = jnp.zeros(
            (L,), jnp.float32
        )
        return c

    lax.fori_loop(0, ZR * (U // L), zbody, 0)
    for i in range(RPT // ZR):
        pltpu.sync_copy(zrows, out_sh.at[pl.ds(sid * RPT + i * ZR, ZR)])
    plsc.subcore_barrier()

    base0 = wid * EPW
    lanes = lax.iota(jnp.int32, L)

    def chunk(t, c):
        base = base0 + t * CH
        pltpu.sync_copy(recv_h.at[pl.ds(base, CH)], ridx)
        pltpu.sync_copy(send_h.at[pl.ds(base, CH)], sidx)
        cp1 = pltpu.async_copy(wn_h.at[sidx], rows, sem1)
        cp2 = pltpu.async_copy(p0_h.at[ridx], s0v, sem2)
        cp3 = pltpu.async_copy(p1_h.at[ridx], s1v, sem3)
        pltpu.sync_copy(ea_h.at[pl.ds(base, CH)], eav)
        cp2.wait()
        cp3.wait()

        def ab(k, cc):
            s = pl.ds(k * L, L)
            eav[s] = eav[s] / jnp.maximum(s0v[s] + s1v[s], 1e-16)
            return cc

        lax.fori_loop(0, CH // L, ab, 0)
        cp1.wait()

        def edge_group(g, cc):
            av = eav[pl.ds(g * L, L)]
            for j in range(L):
                a = av[j]
                e = g * L + j
                for k in range(U // L):
                    s = pl.ds(k * L, L)
                    rows[e, s] = rows[e, s] * a
            return cc

        lax.fori_loop(0, CH // L, edge_group, 0)
        pltpu.sync_copy(rows, out_sh.at[ridx], add=True)
        return c

    lax.fori_loop(0, NCH, chunk, 0)
    plsc.subcore_barrier()
    for i in range(RPT // ZR):
        r = sid * RPT + i * ZR
        pltpu.sync_copy(out_sh.at[pl.ds(r, ZR)], pout_h.at[cid, pl.ds(r, ZR)])


_passB = pl.kernel(
    _passB_body,
    out_type=jax.ShapeDtypeStruct((NC, NPAD, U), jnp.float32),
    mesh=_mesh,
    scratch_types=[
        pltpu.VMEM((CH,), jnp.int32),
        pltpu.VMEM((CH,), jnp.int32),
        pltpu.VMEM((CH,), jnp.float32),
        pltpu.VMEM((CH,), jnp.float32),
        pltpu.VMEM((CH,), jnp.float32),
        pltpu.VMEM((CH, U), jnp.float32),
        pltpu.VMEM((ZR, U), jnp.float32),
        pltpu.VMEM_SHARED((NPAD, U), jnp.float32),
        pltpu.SemaphoreType.DMA,
        pltpu.SemaphoreType.DMA,
        pltpu.SemaphoreType.DMA,
    ],
)


def _combine_body(p_ref, out_ref):
    x = p_ref[0] + p_ref[1]
    out_ref[...] = jnp.where(x > 0, x, jnp.exp(x) - 1.0)


_combine = pl.pallas_call(
    _combine_body,
    grid=(N // _TBLK,),
    in_specs=[pl.BlockSpec((NC, _TBLK, U), lambda i: (0, i, 0))],
    out_specs=pl.BlockSpec((_TBLK, U), lambda i: (i, 0)),
    out_shape=jax.ShapeDtypeStruct((N, U), jnp.float32),
)


def kernel(node, edge, edge_index, W_lin, b_lin, W_att, b_att, w_alpha):
    recv = edge_index[:, 0]
    send = edge_index[:, 1]
    wcat = jnp.concatenate([W_lin, W_att[:F], W_att[F:]], axis=1)
    bcat = jnp.stack([b_lin, b_att])
    wn, sin, sout = _tables(node, wcat, bcat)
    ea, psum = _passA(recv, send, sin, sout, w_alpha[:, 0])
    pout = _passB(recv, send, ea, psum[:NPAD], psum[NPAD:], wn)
    return _combine(pout[:, :N, :])


# speedup vs baseline: 9.3991x; 1.1193x over previous
import jax
import jax.numpy as jnp
from jax import lax
from jax.experimental import pallas as pl
from jax.experimental.pallas import tpu as pltpu
from jax.experimental.pallas import tpu_sc as plsc

N = 10000
M = 320000
F = 128
U = 128

_BITREV = (0, 8, 4, 12, 2, 10, 6, 14, 1, 9, 5, 13, 3, 11, 7, 15)


def _shuffle(v, idx):
    return lax.gather(
        v,
        idx[:, None],
        lax.GatherDimensionNumbers(
            offset_dims=(), collapsed_slice_dims=(0,), start_index_map=(0,)
        ),
        (1,),
        mode=lax.GatherScatterMode.PROMISE_IN_BOUNDS,
    )

NC, NS, L = 2, 16, 16
NW = NC * NS
EPW = M // NW
CH = 80
NCH = EPW // CH
NPAD = 10240
ZCH = NPAD // NS
RPT = NPAD // NS
ZR = 128

_mesh = plsc.VectorSubcoreMesh(
    core_axis_name="c", subcore_axis_name="s", num_cores=NC, num_subcores=NS
)


def _tables_body(x_ref, wcat_ref, bcat_ref, wn_ref, sin_ref, sout_ref):
    x = x_ref[...]
    wn_ref[...] = (
        jnp.dot(x, wcat_ref[:, 0:U], preferred_element_type=jnp.float32)
        + bcat_ref[0, :]
    )
    sin_ref[...] = jnp.dot(x, wcat_ref[:, U : 2 * U], preferred_element_type=jnp.float32)
    sout_ref[...] = (
        jnp.dot(x, wcat_ref[:, 2 * U : 3 * U], preferred_element_type=jnp.float32)
        + bcat_ref[1, :]
    )


_TBLK = 2000
_tables = pl.pallas_call(
    _tables_body,
    grid=(N // _TBLK,),
    in_specs=[
        pl.BlockSpec((_TBLK, F), lambda i: (i, 0)),
        pl.BlockSpec((F, 3 * U), lambda i: (0, 0)),
        pl.BlockSpec((2, U), lambda i: (0, 0)),
    ],
    out_specs=[pl.BlockSpec((_TBLK, U), lambda i: (i, 0))] * 3,
    out_shape=[jax.ShapeDtypeStruct((N, U), jnp.float32)] * 3,
)


def _passA_body(
    recv_h, send_h, sin_h, sout_h, wal_h,
    ea_h, psum_h,
    ridx, sidx, rin, rout, eav, wv, zv, asum_sh, sem1, sem2,
):
    cid = lax.axis_index("c")
    sid = lax.axis_index("s")
    wid = sid * NC + cid
    pltpu.sync_copy(wal_h, wv)

    def zbody(i, c):
        zv[pl.ds(i * L, L)] = jnp.zeros((L,), jnp.float32)
        return c

    lax.fori_loop(0, ZCH // L, zbody, 0)
    pltpu.sync_copy(zv, asum_sh.at[pl.ds(sid * ZCH, ZCH)])
    plsc.subcore_barrier()

    base0 = wid * EPW
    lanes = lax.iota(jnp.int32, L)

    def chunk(t, c):
        base = base0 + t * CH
        pltpu.sync_copy(recv_h.at[pl.ds(base, CH)], ridx)
        pltpu.sync_copy(send_h.at[pl.ds(base, CH)], sidx)
        cp1 = pltpu.async_copy(sin_h.at[ridx], rin, sem1)
        cp2 = pltpu.async_copy(sout_h.at[sidx], rout, sem2)
        cp1.wait()
        cp2.wait()

        def edge_group(g, cc):
            accs = []
            for j in _BITREV:
                e = g * L + j
                acc = jnp.zeros((L,), jnp.float32)
                for k in range(U // L):
                    s = pl.ds(k * L, L)
                    z = rin[e, s] + rout[e, s]
                    z = jnp.maximum(z, 0.2 * z)
                    acc = acc + z * wv[s]
                accs.append(acc)
            sh = 8
            while len(accs) > 1:
                nxt = []
                for i in range(0, len(accs), 2):
                    ta = accs[i] + _shuffle(accs[i], lanes ^ sh)
                    tb = accs[i + 1] + _shuffle(accs[i + 1], lanes ^ sh)
                    nxt.append(jnp.where((lanes & sh) == 0, ta, tb))
                accs = nxt
                sh //= 2
            eav[pl.ds(g * L, L)] = jnp.exp(accs[0])
            return cc

        lax.fori_loop(0, CH // L, edge_group, 0)
        pltpu.sync_copy(eav, ea_h.at[pl.ds(base, CH)])
        pltpu.sync_copy(eav, asum_sh.at[ridx], add=True)
        return c

    lax.fori_loop(0, NCH, chunk, 0)
    plsc.subcore_barrier()

    @pl.when(sid == 0)
    def _():
        pltpu.sync_copy(asum_sh, psum_h.at[pl.ds(cid * NPAD, NPAD)])


_passA = pl.kernel(
    _passA_body,
    out_type=[
        jax.ShapeDtypeStruct((M,), jnp.float32),
        jax.ShapeDtypeStruct((NC * NPAD,), jnp.float32),
    ],
    mesh=_mesh,
    scratch_types=[
        pltpu.VMEM((CH,), jnp.int32),
        pltpu.VMEM((CH,), jnp.int32),
        pltpu.VMEM((CH, U), jnp.float32),
        pltpu.VMEM((CH, U), jnp.float32),
        pltpu.VMEM((CH,), jnp.float32),
        pltpu.VMEM((U,), jnp.float32),
        pltpu.VMEM((ZCH,), jnp.float32),
        pltpu.VMEM_SHARED((NPAD,), jnp.float32),
        pltpu.SemaphoreType.DMA,
        pltpu.SemaphoreType.DMA,
    ],
)


def _passB_body(
    recv_h, send_h, ea_h, p0_h, p1_h, wn_h,
    pout_h,
    ridx, sidx, eav, s0v, s1v, rows, zrows, out_sh, sem1, sem2, sem3,
):
    cid = lax.axis_index("c")
    sid = lax.axis_index("s")
    wid = sid * NC + cid

    def zbody(i, c):
        zrows[i // (U // L), pl.ds((i % (U // L)) * L, L)] = jnp.zeros(
            (L,), jnp.float32
        )
        return c

    lax.fori_loop(0, ZR * (U // L), zbody, 0)
    for i in range(RPT // ZR):
        pltpu.sync_copy(zrows, out_sh.at[pl.ds(sid * RPT + i * ZR, ZR)])
    plsc.subcore_barrier()

    base0 = wid * EPW
    lanes = lax.iota(jnp.int32, L)

    def chunk(t, c):
        base = base0 + t * CH
        pltpu.sync_copy(recv_h.at[pl.ds(base, CH)], ridx)
        pltpu.sync_copy(send_h.at[pl.ds(base, CH)], sidx)
        cp1 = pltpu.async_copy(wn_h.at[sidx], rows, sem1)
        cp2 = pltpu.async_copy(p0_h.at[ridx], s0v, sem2)
        cp3 = pltpu.async_copy(p1_h.at[ridx], s1v, sem3)
        pltpu.sync_copy(ea_h.at[pl.ds(base, CH)], eav)
        cp2.wait()
        cp3.wait()

        def ab(k, cc):
            s = pl.ds(k * L, L)
            eav[s] = eav[s] / jnp.maximum(s0v[s] + s1v[s], 1e-16)
            return cc

        lax.fori_loop(0, CH // L, ab, 0)
        cp1.wait()

        def edge_group(g, cc):
            av = eav[pl.ds(g * L, L)]
            for j in range(L):
                a = av[j]
                e = g * L + j
                for k in range(U // L):
                    s = pl.ds(k * L, L)
                    rows[e, s] = rows[e, s] * a
            return cc

        lax.fori_loop(0, CH // L, edge_group, 0)
        pltpu.sync_copy(rows, out_sh.at[ridx], add=True)
        return c

    lax.fori_loop(0, NCH, chunk, 0)
    plsc.subcore_barrier()
    for i in range(RPT // ZR):
        r = sid * RPT + i * ZR
        pltpu.sync_copy(out_sh.at[pl.ds(r, ZR)], pout_h.at[cid, pl.ds(r, ZR)])


_passB = pl.kernel(
    _passB_body,
    out_type=jax.ShapeDtypeStruct((NC, NPAD, U), jnp.float32),
    mesh=_mesh,
    scratch_types=[
        pltpu.VMEM((CH,), jnp.int32),
        pltpu.VMEM((CH,), jnp.int32),
        pltpu.VMEM((CH,), jnp.float32),
        pltpu.VMEM((CH,), jnp.float32),
        pltpu.VMEM((CH,), jnp.float32),
        pltpu.VMEM((CH, U), jnp.float32),
        pltpu.VMEM((ZR, U), jnp.float32),
        pltpu.VMEM_SHARED((NPAD, U), jnp.float32),
        pltpu.SemaphoreType.DMA,
        pltpu.SemaphoreType.DMA,
        pltpu.SemaphoreType.DMA,
    ],
)


def _combine_body(p_ref, out_ref):
    x = p_ref[0] + p_ref[1]
    out_ref[...] = jnp.where(x > 0, x, jnp.exp(x) - 1.0)


_combine = pl.pallas_call(
    _combine_body,
    grid=(N // _TBLK,),
    in_specs=[pl.BlockSpec((NC, _TBLK, U), lambda i: (0, i, 0))],
    out_specs=pl.BlockSpec((_TBLK, U), lambda i: (i, 0)),
    out_shape=jax.ShapeDtypeStruct((N, U), jnp.float32),
)


def kernel(node, edge, edge_index, W_lin, b_lin, W_att, b_att, w_alpha):
    recv = edge_index[:, 0]
    send = edge_index[:, 1]
    wcat = jnp.concatenate([W_lin, W_att[:F], W_att[F:]], axis=1)
    bcat = jnp.stack([b_lin, b_att])
    wn, sin, sout = _tables(node, wcat, bcat)
    ea, psum = _passA(recv, send, sin, sout, w_alpha[:, 0])
    pout = _passB(recv, send, ea, psum[:NPAD], psum[NPAD:], wn)
    return _combine(pout[:, :N, :])


# speedup vs baseline: 16.4276x; 1.7478x over previous
import jax
import jax.numpy as jnp
from jax import lax
from jax.experimental import pallas as pl
from jax.experimental.pallas import tpu as pltpu
from jax.experimental.pallas import tpu_sc as plsc

N = 10000
M = 320000
F = 128
U = 128

_BITREV = (0, 8, 4, 12, 2, 10, 6, 14, 1, 9, 5, 13, 3, 11, 7, 15)


def _shuffle(v, idx):
    return lax.gather(
        v,
        idx[:, None],
        lax.GatherDimensionNumbers(
            offset_dims=(), collapsed_slice_dims=(0,), start_index_map=(0,)
        ),
        (1,),
        mode=lax.GatherScatterMode.PROMISE_IN_BOUNDS,
    )

NC, NS, L = 2, 16, 16
NW = NC * NS
EPW = M // NW
CH = 80
NCH = EPW // CH
NPAD = 10240
ZCH = NPAD // NS
RPT = NPAD // NS
ZR = 32

_mesh = plsc.VectorSubcoreMesh(
    core_axis_name="c", subcore_axis_name="s", num_cores=NC, num_subcores=NS
)


def _tables_body(x_ref, wcat_ref, bcat_ref, wn_ref, sin_ref, sout_ref):
    x = x_ref[...]
    wn_ref[...] = (
        jnp.dot(x, wcat_ref[:, 0:U], preferred_element_type=jnp.float32)
        + bcat_ref[0, :]
    )
    sin_ref[...] = jnp.dot(x, wcat_ref[:, U : 2 * U], preferred_element_type=jnp.float32)
    sout_ref[...] = (
        jnp.dot(x, wcat_ref[:, 2 * U : 3 * U], preferred_element_type=jnp.float32)
        + bcat_ref[1, :]
    )


_TBLK = 2000
_tables = pl.pallas_call(
    _tables_body,
    grid=(N // _TBLK,),
    in_specs=[
        pl.BlockSpec((_TBLK, F), lambda i: (i, 0)),
        pl.BlockSpec((F, 3 * U), lambda i: (0, 0)),
        pl.BlockSpec((2, U), lambda i: (0, 0)),
    ],
    out_specs=[pl.BlockSpec((_TBLK, U), lambda i: (i, 0))] * 3,
    out_shape=[jax.ShapeDtypeStruct((N, U), jnp.float32)] * 3,
)


def _passA_body(
    recv_h, send_h, sin_h, sout_h, wal_h,
    ea_h, psum_h,
    ridx2, sidx2, rin0, rout0, rin1, rout1, eava, wv, zv, asum_sh,
    semA, semB,
):
    cid = lax.axis_index("c")
    sid = lax.axis_index("s")
    wid = sid * NC + cid
    pltpu.sync_copy(wal_h, wv)

    def zbody(i, c):
        zv[pl.ds(i * L, L)] = jnp.zeros((L,), jnp.float32)
        return c

    lax.fori_loop(0, ZCH // L, zbody, 0)
    pltpu.sync_copy(zv, asum_sh.at[pl.ds(sid * ZCH, ZCH)])

    pltpu.sync_copy(recv_h.at[wid], ridx2)
    pltpu.sync_copy(send_h.at[wid], sidx2)
    plsc.subcore_barrier()

    lanes = lax.iota(jnp.int32, L)

    def issue(t, rin_b, rout_b, sem):
        pltpu.async_copy(sin_h.at[ridx2.at[t]], rin_b, sem)
        pltpu.async_copy(sout_h.at[sidx2.at[t]], rout_b, sem)

    def drain(rin_b, rout_b, sem):
        pltpu.make_async_copy(sin_h.at[ridx2.at[0]], rin_b, sem).wait()
        pltpu.make_async_copy(sout_h.at[sidx2.at[0]], rout_b, sem).wait()

    def compute(t, rin, rout):
        def edge_group(g, cc):
            accs = []
            for j in _BITREV:
                e = g * L + j
                acc = jnp.zeros((L,), jnp.float32)
                for k in range(U // L):
                    s = pl.ds(k * L, L)
                    z = rin[e, s] + rout[e, s]
                    z = jnp.maximum(z, 0.2 * z)
                    acc = acc + z * wv[s]
                accs.append(acc)
            sh = 8
            while len(accs) > 1:
                nxt = []
                for i in range(0, len(accs), 2):
                    ta = accs[i] + _shuffle(accs[i], lanes ^ sh)
                    tb = accs[i + 1] + _shuffle(accs[i + 1], lanes ^ sh)
                    nxt.append(jnp.where((lanes & sh) == 0, ta, tb))
                accs = nxt
                sh //= 2
            eava[t, pl.ds(g * L, L)] = jnp.exp(accs[0])
            return cc

        lax.fori_loop(0, CH // L, edge_group, 0)
        pltpu.sync_copy(eava.at[t], asum_sh.at[ridx2.at[t]], add=True)

    issue(0, rin0, rout0, semA)

    def pair(tt, c):
        a = 2 * tt
        issue(a + 1, rin1, rout1, semB)
        drain(rin0, rout0, semA)
        compute(a, rin0, rout0)
        issue(a + 2, rin0, rout0, semA)
        drain(rin1, rout1, semB)
        compute(a + 1, rin1, rout1)
        return c

    lax.fori_loop(0, (NCH - 1) // 2, pair, 0)
    drain(rin0, rout0, semA)
    compute(NCH - 1, rin0, rout0)

    pltpu.sync_copy(eava, ea_h.at[wid])
    plsc.subcore_barrier()

    @pl.when(sid == 0)
    def _():
        pltpu.sync_copy(asum_sh, psum_h.at[pl.ds(cid * NPAD, NPAD)])


_passA = pl.kernel(
    _passA_body,
    out_type=[
        jax.ShapeDtypeStruct((NW, NCH, CH), jnp.float32),
        jax.ShapeDtypeStruct((NC * NPAD,), jnp.float32),
    ],
    mesh=_mesh,
    scratch_types=[
        pltpu.VMEM((NCH, CH), jnp.int32),
        pltpu.VMEM((NCH, CH), jnp.int32),
        pltpu.VMEM((CH, U), jnp.float32),
        pltpu.VMEM((CH, U), jnp.float32),
        pltpu.VMEM((CH, U), jnp.float32),
        pltpu.VMEM((CH, U), jnp.float32),
        pltpu.VMEM((NCH, CH), jnp.float32),
        pltpu.VMEM((U,), jnp.float32),
        pltpu.VMEM((ZCH,), jnp.float32),
        pltpu.VMEM_SHARED((NPAD,), jnp.float32),
        pltpu.SemaphoreType.DMA,
        pltpu.SemaphoreType.DMA,
    ],
)


def _passB_body(
    recvf_h, sendf_h, ea_h, p0_h, p1_h, wn_h,
    pout_h,
    ridx0, sidx0, ridx1, sidx1, eava, s0a, s1a, s0b, s1b,
    rows0, rows1, zrows, out_sh,
    semA, semB,
):
    cid = lax.axis_index("c")
    sid = lax.axis_index("s")
    wid = sid * NC + cid

    def zbody(i, c):
        zrows[i // (U // L), pl.ds((i % (U // L)) * L, L)] = jnp.zeros(
            (L,), jnp.float32
        )
        return c

    lax.fori_loop(0, ZR * (U // L), zbody, 0)
    for i in range(RPT // ZR):
        pltpu.sync_copy(zrows, out_sh.at[pl.ds(sid * RPT + i * ZR, ZR)])

    pltpu.sync_copy(ea_h.at[wid], eava)
    plsc.subcore_barrier()

    base0 = wid * EPW

    def load_idx(t, ridx_b, sidx_b):
        pltpu.sync_copy(recvf_h.at[pl.ds(base0 + t * CH, CH)], ridx_b)
        pltpu.sync_copy(sendf_h.at[pl.ds(base0 + t * CH, CH)], sidx_b)

    def issue(ridx_b, sidx_b, rows_b, s0_b, s1_b, sem):
        pltpu.async_copy(wn_h.at[sidx_b], rows_b, sem)
        pltpu.async_copy(p0_h.at[ridx_b], s0_b, sem)
        pltpu.async_copy(p1_h.at[ridx_b], s1_b, sem)

    def drain(ridx_b, sidx_b, rows_b, s0_b, s1_b, sem):
        pltpu.make_async_copy(wn_h.at[sidx_b], rows_b, sem).wait()
        pltpu.make_async_copy(p0_h.at[ridx_b], s0_b, sem).wait()
        pltpu.make_async_copy(p1_h.at[ridx_b], s1_b, sem).wait()

    def scale_scatter(t, ridx_b, rows, s0_b, s1_b):
        def ab(k, cc):
            s = pl.ds(k * L, L)
            eava[t, s] = eava[t, s] / jnp.maximum(s0_b[s] + s1_b[s], 1e-16)
            return cc

        lax.fori_loop(0, CH // L, ab, 0)

        def edge_group(g, cc):
            av = eava[t, pl.ds(g * L, L)]
            for j in range(L):
                a = av[j]
                e = g * L + j
                for k in range(U // L):
                    s = pl.ds(k * L, L)
                    rows[e, s] = rows[e, s] * a
            return cc

        lax.fori_loop(0, CH // L, edge_group, 0)
        pltpu.sync_copy(rows, out_sh.at[ridx_b], add=True)

    load_idx(0, ridx0, sidx0)
    issue(ridx0, sidx0, rows0, s0a, s1a, semA)

    def pair(tt, c):
        a = 2 * tt
        load_idx(a + 1, ridx1, sidx1)
        issue(ridx1, sidx1, rows1, s0b, s1b, semB)
        drain(ridx0, sidx0, rows0, s0a, s1a, semA)
        scale_scatter(a, ridx0, rows0, s0a, s1a)
        load_idx(a + 2, ridx0, sidx0)
        issue(ridx0, sidx0, rows0, s0a, s1a, semA)
        drain(ridx1, sidx1, rows1, s0b, s1b, semB)
        scale_scatter(a + 1, ridx1, rows1, s0b, s1b)
        return c

    lax.fori_loop(0, (NCH - 1) // 2, pair, 0)
    drain(ridx0, sidx0, rows0, s0a, s1a, semA)
    scale_scatter(NCH - 1, ridx0, rows0, s0a, s1a)

    plsc.subcore_barrier()
    for i in range(RPT // ZR):
        r = sid * RPT + i * ZR
        pltpu.sync_copy(out_sh.at[pl.ds(r, ZR)], pout_h.at[cid, pl.ds(r, ZR)])


_passB = pl.kernel(
    _passB_body,
    out_type=jax.ShapeDtypeStruct((NC, NPAD, U), jnp.float32),
    mesh=_mesh,
    scratch_types=[
        pltpu.VMEM((CH,), jnp.int32),
        pltpu.VMEM((CH,), jnp.int32),
        pltpu.VMEM((CH,), jnp.int32),
        pltpu.VMEM((CH,), jnp.int32),
        pltpu.VMEM((NCH, CH), jnp.float32),
        pltpu.VMEM((CH,), jnp.float32),
        pltpu.VMEM((CH,), jnp.float32),
        pltpu.VMEM((CH,), jnp.float32),
        pltpu.VMEM((CH,), jnp.float32),
        pltpu.VMEM((CH, U), jnp.float32),
        pltpu.VMEM((CH, U), jnp.float32),
        pltpu.VMEM((ZR, U), jnp.float32),
        pltpu.VMEM_SHARED((NPAD, U), jnp.float32),
        pltpu.SemaphoreType.DMA,
        pltpu.SemaphoreType.DMA,
    ],
)


def _combine_body(p_ref, out_ref):
    x = p_ref[0] + p_ref[1]
    out_ref[...] = jnp.where(x > 0, x, jnp.exp(x) - 1.0)


_combine = pl.pallas_call(
    _combine_body,
    grid=(N // _TBLK,),
    in_specs=[pl.BlockSpec((NC, _TBLK, U), lambda i: (0, i, 0))],
    out_specs=pl.BlockSpec((_TBLK, U), lambda i: (i, 0)),
    out_shape=jax.ShapeDtypeStruct((N, U), jnp.float32),
)


def kernel(node, edge, edge_index, W_lin, b_lin, W_att, b_att, w_alpha):
    recvf = edge_index[:, 0]
    sendf = edge_index[:, 1]
    recv3 = recvf.reshape(NW, NCH, CH)
    send3 = sendf.reshape(NW, NCH, CH)
    wcat = jnp.concatenate([W_lin, W_att[:F], W_att[F:]], axis=1)
    bcat = jnp.stack([b_lin, b_att])
    wn, sin, sout = _tables(node, wcat, bcat)
    ea, psum = _passA(recv3, send3, sin, sout, w_alpha[:, 0])
    pout = _passB(recvf, sendf, ea, psum[:NPAD], psum[NPAD:], wn)
    return _combine(pout[:, :N, :])


# speedup vs baseline: 16.6574x; 1.0140x over previous
import jax
import jax.numpy as jnp
from jax import lax
from jax.experimental import pallas as pl
from jax.experimental.pallas import tpu as pltpu
from jax.experimental.pallas import tpu_sc as plsc

N = 10000
M = 320000
F = 128
U = 128

_BITREV = (0, 8, 4, 12, 2, 10, 6, 14, 1, 9, 5, 13, 3, 11, 7, 15)


def _shuffle(v, idx):
    return lax.gather(
        v,
        idx[:, None],
        lax.GatherDimensionNumbers(
            offset_dims=(), collapsed_slice_dims=(0,), start_index_map=(0,)
        ),
        (1,),
        mode=lax.GatherScatterMode.PROMISE_IN_BOUNDS,
    )

NC, NS, L = 2, 16, 16
NW = NC * NS
EPW = M // NW
CH = 80
NCH = EPW // CH
NPAD = 10240
ZCH = NPAD // NS
RPT = NPAD // NS
ZR = 32

_mesh = plsc.VectorSubcoreMesh(
    core_axis_name="c", subcore_axis_name="s", num_cores=NC, num_subcores=NS
)


def _tables_body(x_ref, wcat_ref, bcat_ref, wn_ref, sin_ref, sout_ref):
    x = x_ref[...]
    wn_ref[...] = (
        jnp.dot(x, wcat_ref[:, 0:U], preferred_element_type=jnp.float32)
        + bcat_ref[0, :]
    )
    sin_ref[...] = jnp.dot(x, wcat_ref[:, U : 2 * U], preferred_element_type=jnp.float32)
    sout_ref[...] = (
        jnp.dot(x, wcat_ref[:, 2 * U : 3 * U], preferred_element_type=jnp.float32)
        + bcat_ref[1, :]
    )


_TBLK = 2000
_tables = pl.pallas_call(
    _tables_body,
    grid=(N // _TBLK,),
    in_specs=[
        pl.BlockSpec((_TBLK, F), lambda i: (i, 0)),
        pl.BlockSpec((F, 3 * U), lambda i: (0, 0)),
        pl.BlockSpec((2, U), lambda i: (0, 0)),
    ],
    out_specs=[pl.BlockSpec((_TBLK, U), lambda i: (i, 0))] * 3,
    out_shape=[jax.ShapeDtypeStruct((N, U), jnp.float32)] * 3,
)


def _passA_body(
    recv_h, send_h, sin_h, sout_h, wal_h,
    ea_h, psum_h,
    ridx2, sidx2, rin0, rout0, rin1, rout1, eava, wv, zv, asum_sh,
    semA, semB, semC,
):
    cid = lax.axis_index("c")
    sid = lax.axis_index("s")
    wid = sid * NC + cid
    pltpu.sync_copy(wal_h, wv)

    def zbody(i, c):
        zv[pl.ds(i * L, L)] = jnp.zeros((L,), jnp.float32)
        return c

    lax.fori_loop(0, ZCH // L, zbody, 0)
    pltpu.sync_copy(zv, asum_sh.at[pl.ds(sid * ZCH, ZCH)])

    pltpu.sync_copy(recv_h.at[wid], ridx2)
    pltpu.sync_copy(send_h.at[wid], sidx2)
    plsc.subcore_barrier()

    lanes = lax.iota(jnp.int32, L)

    def issue(t, rin_b, rout_b, sem):
        pltpu.async_copy(sin_h.at[ridx2.at[t]], rin_b, sem)
        pltpu.async_copy(sout_h.at[sidx2.at[t]], rout_b, sem)

    def drain(rin_b, rout_b, sem):
        pltpu.make_async_copy(sin_h.at[ridx2.at[0]], rin_b, sem).wait()
        pltpu.make_async_copy(sout_h.at[sidx2.at[0]], rout_b, sem).wait()

    def compute(t, rin, rout):
        def edge_group(g, cc):
            accs = []
            for j in _BITREV:
                e = g * L + j
                acc = jnp.zeros((L,), jnp.float32)
                for k in range(U // L):
                    s = pl.ds(k * L, L)
                    z = rin[e, s] + rout[e, s]
                    z = jnp.maximum(z, 0.2 * z)
                    acc = acc + z * wv[s]
                accs.append(acc)
            sh = 8
            while len(accs) > 1:
                nxt = []
                for i in range(0, len(accs), 2):
                    ta = accs[i] + _shuffle(accs[i], lanes ^ sh)
                    tb = accs[i + 1] + _shuffle(accs[i + 1], lanes ^ sh)
                    nxt.append(jnp.where((lanes & sh) == 0, ta, tb))
                accs = nxt
                sh //= 2
            eava[t, pl.ds(g * L, L)] = jnp.exp(accs[0])
            return cc

        lax.fori_loop(0, CH // L, edge_group, 0)
        pltpu.async_copy(eava.at[t], asum_sh.at[ridx2.at[t]], semC, add=True)

    issue(0, rin0, rout0, semA)

    def pair(tt, c):
        a = 2 * tt
        issue(a + 1, rin1, rout1, semB)
        drain(rin0, rout0, semA)
        compute(a, rin0, rout0)
        issue(a + 2, rin0, rout0, semA)
        drain(rin1, rout1, semB)
        compute(a + 1, rin1, rout1)
        return c

    lax.fori_loop(0, (NCH - 1) // 2, pair, 0)
    drain(rin0, rout0, semA)
    compute(NCH - 1, rin0, rout0)

    def drain_sc(t, c):
        pltpu.make_async_copy(
            eava.at[0], asum_sh.at[ridx2.at[0]], semC
        ).wait()
        return c

    lax.fori_loop(0, NCH, drain_sc, 0)
    pltpu.sync_copy(eava, ea_h.at[wid])
    plsc.subcore_barrier()

    @pl.when(sid == 0)
    def _():
        pltpu.sync_copy(asum_sh, psum_h.at[pl.ds(cid * NPAD, NPAD)])


_passA = pl.kernel(
    _passA_body,
    out_type=[
        jax.ShapeDtypeStruct((NW, NCH, CH), jnp.float32),
        jax.ShapeDtypeStruct((NC * NPAD,), jnp.float32),
    ],
    mesh=_mesh,
    scratch_types=[
        pltpu.VMEM((NCH, CH), jnp.int32),
        pltpu.VMEM((NCH, CH), jnp.int32),
        pltpu.VMEM((CH, U), jnp.float32),
        pltpu.VMEM((CH, U), jnp.float32),
        pltpu.VMEM((CH, U), jnp.float32),
        pltpu.VMEM((CH, U), jnp.float32),
        pltpu.VMEM((NCH, CH), jnp.float32),
        pltpu.VMEM((U,), jnp.float32),
        pltpu.VMEM((ZCH,), jnp.float32),
        pltpu.VMEM_SHARED((NPAD,), jnp.float32),
        pltpu.SemaphoreType.DMA,
        pltpu.SemaphoreType.DMA,
        pltpu.SemaphoreType.DMA,
    ],
)


def _passB_body(
    recvf_h, sendf_h, ea_h, p0_h, p1_h, wn_h,
    pout_h,
    ridx0, sidx0, ridx1, sidx1, eava, invv, p0v, p1v,
    rows0, rows1, zrows, out_sh,
    semA, semB,
):
    cid = lax.axis_index("c")
    sid = lax.axis_index("s")
    wid = sid * NC + cid

    def zbody(i, c):
        zrows[i // (U // L), pl.ds((i % (U // L)) * L, L)] = jnp.zeros(
            (L,), jnp.float32
        )
        return c

    lax.fori_loop(0, ZR * (U // L), zbody, 0)
    for i in range(RPT // ZR):
        pltpu.sync_copy(zrows, out_sh.at[pl.ds(sid * RPT + i * ZR, ZR)])

    pltpu.sync_copy(ea_h.at[wid], eava)
    pltpu.sync_copy(p0_h.at[pl.ds(sid * RPT, RPT)], p0v)
    pltpu.sync_copy(p1_h.at[pl.ds(sid * RPT, RPT)], p1v)

    def invb(i, c):
        s = pl.ds(i * L, L)
        invv[s] = 1.0 / jnp.maximum(p0v[s] + p1v[s], 1e-16)
        return c

    lax.fori_loop(0, RPT // L, invb, 0)
    plsc.subcore_barrier()

    base0 = wid * EPW

    def load_idx(t, ridx_b, sidx_b):
        pltpu.sync_copy(recvf_h.at[pl.ds(base0 + t * CH, CH)], ridx_b)
        pltpu.sync_copy(sendf_h.at[pl.ds(base0 + t * CH, CH)], sidx_b)

    def issue(sidx_b, rows_b, sem):
        pltpu.async_copy(wn_h.at[sidx_b], rows_b, sem)

    def drain(sidx_b, rows_b, sem):
        pltpu.make_async_copy(wn_h.at[sidx_b], rows_b, sem).wait()

    def scale_scatter(t, ridx_b, rows):
        def edge_group(g, cc):
            av = eava[t, pl.ds(g * L, L)]
            for j in range(L):
                a = av[j]
                e = g * L + j
                for k in range(U // L):
                    s = pl.ds(k * L, L)
                    rows[e, s] = rows[e, s] * a
            return cc

        lax.fori_loop(0, CH // L, edge_group, 0)
        pltpu.sync_copy(rows, out_sh.at[ridx_b], add=True)

    load_idx(0, ridx0, sidx0)
    issue(sidx0, rows0, semA)

    def pair(tt, c):
        a = 2 * tt
        load_idx(a + 1, ridx1, sidx1)
        issue(sidx1, rows1, semB)
        drain(sidx0, rows0, semA)
        scale_scatter(a, ridx0, rows0)
        load_idx(a + 2, ridx0, sidx0)
        issue(sidx0, rows0, semA)
        drain(sidx1, rows1, semB)
        scale_scatter(a + 1, ridx1, rows1)
        return c

    lax.fori_loop(0, (NCH - 1) // 2, pair, 0)
    drain(sidx0, rows0, semA)
    scale_scatter(NCH - 1, ridx0, rows0)

    plsc.subcore_barrier()

    for blk in range(RPT // ZR):
        pltpu.sync_copy(out_sh.at[pl.ds(sid * RPT + blk * ZR, ZR)], zrows)

        def rs(i, c):
            g = blk * (ZR // L) + i
            iv = invv[pl.ds(g * L, L)]
            for j in range(L):
                a = iv[j]
                r = (i * L + j) % ZR
                for k in range(U // L):
                    s = pl.ds(k * L, L)
                    zrows[r, s] = zrows[r, s] * a
            return c

        lax.fori_loop(0, ZR // L, rs, 0)
        pltpu.sync_copy(
            zrows, pout_h.at[cid, pl.ds(sid * RPT + blk * ZR, ZR)]
        )


_passB = pl.kernel(
    _passB_body,
    out_type=jax.ShapeDtypeStruct((NC, NPAD, U), jnp.float32),
    mesh=_mesh,
    scratch_types=[
        pltpu.VMEM((CH,), jnp.int32),
        pltpu.VMEM((CH,), jnp.int32),
        pltpu.VMEM((CH,), jnp.int32),
        pltpu.VMEM((CH,), jnp.int32),
        pltpu.VMEM((NCH, CH), jnp.float32),
        pltpu.VMEM((RPT,), jnp.float32),
        pltpu.VMEM((RPT,), jnp.float32),
        pltpu.VMEM((RPT,), jnp.float32),
        pltpu.VMEM((CH, U), jnp.float32),
        pltpu.VMEM((CH, U), jnp.float32),
        pltpu.VMEM((ZR, U), jnp.float32),
        pltpu.VMEM_SHARED((NPAD, U), jnp.float32),
        pltpu.SemaphoreType.DMA,
        pltpu.SemaphoreType.DMA,
    ],
)


def _combine_body(p_ref, out_ref):
    x = p_ref[0] + p_ref[1]
    out_ref[...] = jnp.where(x > 0, x, jnp.exp(x) - 1.0)


_combine = pl.pallas_call(
    _combine_body,
    grid=(N // _TBLK,),
    in_specs=[pl.BlockSpec((NC, _TBLK, U), lambda i: (0, i, 0))],
    out_specs=pl.BlockSpec((_TBLK, U), lambda i: (i, 0)),
    out_shape=jax.ShapeDtypeStruct((N, U), jnp.float32),
)


def kernel(node, edge, edge_index, W_lin, b_lin, W_att, b_att, w_alpha):
    recvf = edge_index[:, 0]
    sendf = edge_index[:, 1]
    recv3 = recvf.reshape(NW, NCH, CH)
    send3 = sendf.reshape(NW, NCH, CH)
    wcat = jnp.concatenate([W_lin, W_att[:F], W_att[F:]], axis=1)
    bcat = jnp.stack([b_lin, b_att])
    wn, sin, sout = _tables(node, wcat, bcat)
    ea, psum = _passA(recv3, send3, sin, sout, w_alpha[:, 0])
    pout = _passB(recvf, sendf, ea, psum[:NPAD], psum[NPAD:], wn)
    return _combine(pout[:, :N, :])


# speedup vs baseline: 16.8431x; 1.0111x over previous
import jax
import jax.numpy as jnp
from jax import lax
from jax.experimental import pallas as pl
from jax.experimental.pallas import tpu as pltpu
from jax.experimental.pallas import tpu_sc as plsc

N = 10000
M = 320000
F = 128
U = 128

_BITREV = (0, 8, 4, 12, 2, 10, 6, 14, 1, 9, 5, 13, 3, 11, 7, 15)


def _shuffle(v, idx):
    return lax.gather(
        v,
        idx[:, None],
        lax.GatherDimensionNumbers(
            offset_dims=(), collapsed_slice_dims=(0,), start_index_map=(0,)
        ),
        (1,),
        mode=lax.GatherScatterMode.PROMISE_IN_BOUNDS,
    )

NC, NS, L = 2, 16, 16
NW = NC * NS
EPW = M // NW
CH = 80
NCH = EPW // CH
NPAD = 10240
ZCH = NPAD // NS
RPT = NPAD // NS
ZR = 32

_mesh = plsc.VectorSubcoreMesh(
    core_axis_name="c", subcore_axis_name="s", num_cores=NC, num_subcores=NS
)


def _tables_body(x_ref, wcat_ref, bcat_ref, wn_ref, sin_ref, sout_ref):
    x = x_ref[...]
    wn_ref[...] = (
        jnp.dot(x, wcat_ref[:, 0:U], preferred_element_type=jnp.float32)
        + bcat_ref[0, :]
    )
    sin_ref[...] = jnp.dot(x, wcat_ref[:, U : 2 * U], preferred_element_type=jnp.float32)
    sout_ref[...] = (
        jnp.dot(x, wcat_ref[:, 2 * U : 3 * U], preferred_element_type=jnp.float32)
        + bcat_ref[1, :]
    )


_TBLK = 2000
_tables = pl.pallas_call(
    _tables_body,
    grid=(N // _TBLK,),
    in_specs=[
        pl.BlockSpec((_TBLK, F), lambda i: (i, 0)),
        pl.BlockSpec((F, 3 * U), lambda i: (0, 0)),
        pl.BlockSpec((2, U), lambda i: (0, 0)),
    ],
    out_specs=[pl.BlockSpec((_TBLK, U), lambda i: (i, 0))] * 3,
    out_shape=[jax.ShapeDtypeStruct((N, U), jnp.float32)] * 3,
)


def _passA_body(
    recv_h, send_h, sin_h, sout_h, wal_h,
    ea_h, psum_h,
    ridx2, sidx2, rin0, rout0, rin1, rout1, eava, wv, zv, asum_sh,
    semA, semB, semC,
):
    cid = lax.axis_index("c")
    sid = lax.axis_index("s")
    wid = sid * NC + cid
    pltpu.sync_copy(wal_h, wv)

    def zbody(i, c):
        zv[pl.ds(i * L, L)] = jnp.zeros((L,), jnp.float32)
        return c

    lax.fori_loop(0, ZCH // L, zbody, 0)
    pltpu.sync_copy(zv, asum_sh.at[pl.ds(sid * ZCH, ZCH)])

    pltpu.sync_copy(recv_h.at[wid], ridx2)
    pltpu.sync_copy(send_h.at[wid], sidx2)
    plsc.subcore_barrier()

    lanes = lax.iota(jnp.int32, L)

    def issue(t, rin_b, rout_b, sem):
        pltpu.async_copy(sin_h.at[ridx2.at[t]], rin_b, sem)
        pltpu.async_copy(sout_h.at[sidx2.at[t]], rout_b, sem)

    def drain(rin_b, rout_b, sem):
        pltpu.make_async_copy(sin_h.at[ridx2.at[0]], rin_b, sem).wait()
        pltpu.make_async_copy(sout_h.at[sidx2.at[0]], rout_b, sem).wait()

    def compute(t, rin, rout):
        def edge_group(g, cc):
            accs = []
            for j in _BITREV:
                e = g * L + j
                acc = jnp.zeros((L,), jnp.float32)
                for k in range(U // L):
                    s = pl.ds(k * L, L)
                    z = rin[e, s] + rout[e, s]
                    z = jnp.maximum(z, 0.2 * z)
                    acc = acc + z * wv[s]
                accs.append(acc)
            sh = 8
            while len(accs) > 1:
                nxt = []
                for i in range(0, len(accs), 2):
                    ta = accs[i] + _shuffle(accs[i], lanes ^ sh)
                    tb = accs[i + 1] + _shuffle(accs[i + 1], lanes ^ sh)
                    nxt.append(jnp.where((lanes & sh) == 0, ta, tb))
                accs = nxt
                sh //= 2
            eava[t, pl.ds(g * L, L)] = jnp.exp(accs[0])
            return cc

        lax.fori_loop(0, CH // L, edge_group, 0)
        pltpu.async_copy(eava.at[t], asum_sh.at[ridx2.at[t]], semC, add=True)

    issue(0, rin0, rout0, semA)

    def pair(tt, c):
        a = 2 * tt
        issue(a + 1, rin1, rout1, semB)
        drain(rin0, rout0, semA)
        compute(a, rin0, rout0)
        issue(a + 2, rin0, rout0, semA)
        drain(rin1, rout1, semB)
        compute(a + 1, rin1, rout1)
        return c

    lax.fori_loop(0, (NCH - 1) // 2, pair, 0)
    drain(rin0, rout0, semA)
    compute(NCH - 1, rin0, rout0)

    def drain_sc(t, c):
        pltpu.make_async_copy(
            eava.at[0], asum_sh.at[ridx2.at[0]], semC
        ).wait()
        return c

    lax.fori_loop(0, NCH, drain_sc, 0)
    pltpu.sync_copy(eava, ea_h.at[wid])
    plsc.subcore_barrier()

    @pl.when(sid == 0)
    def _():
        pltpu.sync_copy(asum_sh, psum_h.at[pl.ds(cid * NPAD, NPAD)])


_passA = pl.kernel(
    _passA_body,
    out_type=[
        jax.ShapeDtypeStruct((NW, NCH, CH), jnp.float32),
        jax.ShapeDtypeStruct((NC * NPAD,), jnp.float32),
    ],
    mesh=_mesh,
    scratch_types=[
        pltpu.VMEM((NCH, CH), jnp.int32),
        pltpu.VMEM((NCH, CH), jnp.int32),
        pltpu.VMEM((CH, U), jnp.float32),
        pltpu.VMEM((CH, U), jnp.float32),
        pltpu.VMEM((CH, U), jnp.float32),
        pltpu.VMEM((CH, U), jnp.float32),
        pltpu.VMEM((NCH, CH), jnp.float32),
        pltpu.VMEM((U,), jnp.float32),
        pltpu.VMEM((ZCH,), jnp.float32),
        pltpu.VMEM_SHARED((NPAD,), jnp.float32),
        pltpu.SemaphoreType.DMA,
        pltpu.SemaphoreType.DMA,
        pltpu.SemaphoreType.DMA,
    ],
)


def _passB_body(
    recvf_h, sendf_h, ea_h, p0_h, p1_h, wn_h,
    pout_h,
    ridx0, sidx0, ridx1, sidx1, eava, invv, p0v, p1v,
    rows0, rows1, zrows, out_sh,
    semA, semB, semS0, semS1,
):
    cid = lax.axis_index("c")
    sid = lax.axis_index("s")
    wid = sid * NC + cid

    def zbody(i, c):
        zrows[i // (U // L), pl.ds((i % (U // L)) * L, L)] = jnp.zeros(
            (L,), jnp.float32
        )
        return c

    lax.fori_loop(0, ZR * (U // L), zbody, 0)
    for i in range(RPT // ZR):
        pltpu.sync_copy(zrows, out_sh.at[pl.ds(sid * RPT + i * ZR, ZR)])

    pltpu.sync_copy(ea_h.at[wid], eava)
    pltpu.sync_copy(p0_h.at[pl.ds(sid * RPT, RPT)], p0v)
    pltpu.sync_copy(p1_h.at[pl.ds(sid * RPT, RPT)], p1v)

    def invb(i, c):
        s = pl.ds(i * L, L)
        invv[s] = 1.0 / jnp.maximum(p0v[s] + p1v[s], 1e-16)
        return c

    lax.fori_loop(0, RPT // L, invb, 0)
    plsc.subcore_barrier()

    base0 = wid * EPW

    def load_idx(t, ridx_b, sidx_b):
        pltpu.sync_copy(recvf_h.at[pl.ds(base0 + t * CH, CH)], ridx_b)
        pltpu.sync_copy(sendf_h.at[pl.ds(base0 + t * CH, CH)], sidx_b)

    def issue(sidx_b, rows_b, sem):
        pltpu.async_copy(wn_h.at[sidx_b], rows_b, sem)

    def drain(sidx_b, rows_b, sem):
        pltpu.make_async_copy(wn_h.at[sidx_b], rows_b, sem).wait()

    def scale(t, rows):
        def edge_group(g, cc):
            av = eava[t, pl.ds(g * L, L)]
            for j in range(L):
                a = av[j]
                e = g * L + j
                for k in range(U // L):
                    s = pl.ds(k * L, L)
                    rows[e, s] = rows[e, s] * a
            return cc

        lax.fori_loop(0, CH // L, edge_group, 0)

    def scat(ridx_b, rows, sem):
        pltpu.async_copy(rows, out_sh.at[ridx_b], sem, add=True)

    def wait_scat(ridx_b, rows, sem):
        pltpu.make_async_copy(rows, out_sh.at[ridx_b], sem).wait()

    def zb2(i, c):
        rows1[i // (U // L), pl.ds((i % (U // L)) * L, L)] = jnp.zeros(
            (L,), jnp.float32
        )
        return c

    lax.fori_loop(0, CH * (U // L), zb2, 0)

    def zi(i, c):
        ridx1[pl.ds(i * L, L)] = jnp.zeros((L,), jnp.int32)
        return c

    lax.fori_loop(0, CH // L, zi, 0)
    scat(ridx1, rows1, semS1)

    load_idx(0, ridx0, sidx0)
    issue(sidx0, rows0, semA)

    def pair(tt, c):
        a = 2 * tt
        wait_scat(ridx1, rows1, semS1)
        load_idx(a + 1, ridx1, sidx1)
        issue(sidx1, rows1, semB)
        drain(sidx0, rows0, semA)
        scale(a, rows0)
        scat(ridx0, rows0, semS0)
        drain(sidx1, rows1, semB)
        scale(a + 1, rows1)
        scat(ridx1, rows1, semS1)
        wait_scat(ridx0, rows0, semS0)
        load_idx(a + 2, ridx0, sidx0)
        issue(sidx0, rows0, semA)
        return c

    lax.fori_loop(0, (NCH - 1) // 2, pair, 0)
    wait_scat(ridx1, rows1, semS1)
    drain(sidx0, rows0, semA)
    scale(NCH - 1, rows0)
    scat(ridx0, rows0, semS0)
    wait_scat(ridx0, rows0, semS0)

    plsc.subcore_barrier()

    for blk in range(RPT // ZR):
        pltpu.sync_copy(out_sh.at[pl.ds(sid * RPT + blk * ZR, ZR)], zrows)

        def rs(i, c):
            g = blk * (ZR // L) + i
            iv = invv[pl.ds(g * L, L)]
            for j in range(L):
                a = iv[j]
                r = (i * L + j) % ZR
                for k in range(U // L):
                    s = pl.ds(k * L, L)
                    zrows[r, s] = zrows[r, s] * a
            return c

        lax.fori_loop(0, ZR // L, rs, 0)
        pltpu.sync_copy(
            zrows, pout_h.at[cid, pl.ds(sid * RPT + blk * ZR, ZR)]
        )


_passB = pl.kernel(
    _passB_body,
    out_type=jax.ShapeDtypeStruct((NC, NPAD, U), jnp.float32),
    mesh=_mesh,
    scratch_types=[
        pltpu.VMEM((CH,), jnp.int32),
        pltpu.VMEM((CH,), jnp.int32),
        pltpu.VMEM((CH,), jnp.int32),
        pltpu.VMEM((CH,), jnp.int32),
        pltpu.VMEM((NCH, CH), jnp.float32),
        pltpu.VMEM((RPT,), jnp.float32),
        pltpu.VMEM((RPT,), jnp.float32),
        pltpu.VMEM((RPT,), jnp.float32),
        pltpu.VMEM((CH, U), jnp.float32),
        pltpu.VMEM((CH, U), jnp.float32),
        pltpu.VMEM((ZR, U), jnp.float32),
        pltpu.VMEM_SHARED((NPAD, U), jnp.float32),
        pltpu.SemaphoreType.DMA,
        pltpu.SemaphoreType.DMA,
        pltpu.SemaphoreType.DMA,
        pltpu.SemaphoreType.DMA,
    ],
)


def _combine_body(p_ref, out_ref):
    x = p_ref[0] + p_ref[1]
    out_ref[...] = jnp.where(x > 0, x, jnp.exp(x) - 1.0)


_combine = pl.pallas_call(
    _combine_body,
    grid=(N // _TBLK,),
    in_specs=[pl.BlockSpec((NC, _TBLK, U), lambda i: (0, i, 0))],
    out_specs=pl.BlockSpec((_TBLK, U), lambda i: (i, 0)),
    out_shape=jax.ShapeDtypeStruct((N, U), jnp.float32),
)


def kernel(node, edge, edge_index, W_lin, b_lin, W_att, b_att, w_alpha):
    recvf = edge_index[:, 0]
    sendf = edge_index[:, 1]
    recv3 = recvf.reshape(NW, NCH, CH)
    send3 = sendf.reshape(NW, NCH, CH)
    wcat = jnp.concatenate([W_lin, W_att[:F], W_att[F:]], axis=1)
    bcat = jnp.stack([b_lin, b_att])
    wn, sin, sout = _tables(node, wcat, bcat)
    ea, psum = _passA(recv3, send3, sin, sout, w_alpha[:, 0])
    pout = _passB(recvf, sendf, ea, psum[:NPAD], psum[NPAD:], wn)
    return _combine(pout[:, :N, :])
